# baseline (speedup 1.0000x reference)
import jax
import jax.numpy as jnp
from jax import lax
from jax.experimental import pallas as pl
from jax.experimental.pallas import tpu as pltpu
from jax.experimental.pallas import tpu_sc as plsc

U = 5000
I = 5000
D = 128
NNZ = 320000
NB = 3

NC = 2
NS = 16
CH = 80
NCH = NNZ // (NS * CH)
EPT = NNZ // NS
RPT = 320
ACC_ROWS = NS * RPT
NBUF = 5


def _sc_spmm(meta_u_hbm, meta_i_hbm,
             utab_hbm, itab_hbm, zeros_hbm,
             ue_out, ie_out,
             mbufs, inbufs, outbufs,
             isems, gsems, ssems, acc):
    c = lax.axis_index("c")
    s = lax.axis_index("s")

    def do_side(src_tab, meta_hbm, out_hbm):
        def behavior_body(b, carry0):
            pltpu.sync_copy(zeros_hbm.at[pl.ds(s * RPT, RPT)],
                            acc.at[pl.ds(s * RPT, RPT)])
            plsc.subcore_barrier()
            cbase = (b * NS + s) * NCH

            def ifetch_start(i, k):
                pltpu.async_copy(meta_hbm.at[cbase + i], mbufs[k], isems[k])

            def ifetch_wait(i, k):
                pltpu.make_async_copy(meta_hbm.at[cbase + i], mbufs[k],
                                      isems[k]).wait()

            def gather_start(k):
                pltpu.async_copy(src_tab.at[mbufs[k].at[0]],
                                 inbufs[k], gsems[k])

            def gather_wait(k):
                pltpu.make_async_copy(src_tab.at[mbufs[k].at[0]],
                                      inbufs[k], gsems[k]).wait()

            def scatter_start(k):
                pltpu.async_copy(outbufs[k], acc.at[mbufs[k].at[1]],
                                 ssems[k], add=True)

            def scatter_wait(k):
                pltpu.make_async_copy(outbufs[k], acc.at[mbufs[k].at[1]],
                                      ssems[k]).wait()

            def scale(k):
                inb = inbufs[k]
                outb = outbufs[k]
                mbuf = mbufs[k]
                mask = jnp.int32(-65536)

                @plsc.parallel_loop(0, CH, unroll=5)
                def _(e):
                    val = lax.bitcast_convert_type(
                        mbuf[2, pl.ds(e, 16)], jnp.float32)[0]
                    for g in range(D // 32):
                        w = inb[e, pl.ds(16 * g, 16)]
                        lo = lax.bitcast_convert_type(
                            lax.shift_left(w, jnp.int32(16)), jnp.float32)
                        hi = lax.bitcast_convert_type(w & mask, jnp.float32)
                        outb[e, pl.ds(32 * g, 16)] = lo * val
                        outb[e, pl.ds(32 * g + 16, 16)] = hi * val

            for k in range(3):
                ifetch_start(k, k)
            for k in range(2):
                ifetch_wait(k, k)
                gather_start(k)

            def chunk_group(g, carry):
                for j in range(NBUF):
                    ch = g * NBUF + j
                    pre = ch + 3
                    kpre = (j + 3) % NBUF
                    mid = ch + 2
                    kmid = (j + 2) % NBUF

                    @pl.when(pre < NCH)
                    def _():
                        @pl.when(pre >= NBUF)
                        def _():
                            scatter_wait(kpre)

                        ifetch_start(pre, kpre)

                    @pl.when(mid < NCH)
                    def _():
                        ifetch_wait(mid, kmid)
                        gather_start(kmid)

                    gather_wait(j)
                    scale(j)
                    scatter_start(j)
                return carry

            lax.fori_loop(0, NCH // NBUF, chunk_group, 0, unroll=False)
            for j in range(NBUF):
                scatter_wait(j)
            plsc.subcore_barrier()

            @pl.when(s < NS - 1)
            def _():
                pltpu.sync_copy(acc.at[pl.ds(s * RPT, RPT)],
                                out_hbm.at[b, pl.ds(s * RPT, RPT)])

            @pl.when(s == NS - 1)
            def _():
                last = U - (NS - 1) * RPT
                pltpu.sync_copy(acc.at[pl.ds((NS - 1) * RPT, last)],
                                out_hbm.at[b, pl.ds((NS - 1) * RPT, last)])

            plsc.subcore_barrier()
            return carry0

        lax.fori_loop(0, NB, behavior_body, 0, unroll=False)

    @pl.when(c == 0)
    def _():
        do_side(itab_hbm, meta_u_hbm, ue_out)

    @pl.when(c == 1)
    def _():
        do_side(utab_hbm, meta_i_hbm, ie_out)


@jax.jit
def _spmm_all(meta_u, meta_i, utab, itab, zeros):
    mesh = plsc.VectorSubcoreMesh(core_axis_name="c", subcore_axis_name="s",
                                  num_cores=NC, num_subcores=NS)
    f = pl.kernel(
        _sc_spmm,
        out_type=(jax.ShapeDtypeStruct((NB, U, D), jnp.float32),
                  jax.ShapeDtypeStruct((NB, I, D), jnp.float32)),
        mesh=mesh,
        compiler_params=pltpu.CompilerParams(use_tc_tiling_on_sc=False),
        scratch_types=[
            [pltpu.VMEM((4, CH), jnp.int32) for _ in range(NBUF)],
            [pltpu.VMEM((CH, D // 2), jnp.int32) for _ in range(NBUF)],
            [pltpu.VMEM((CH, D), jnp.float32) for _ in range(NBUF)],
            [pltpu.SemaphoreType.DMA for _ in range(NBUF)],
            [pltpu.SemaphoreType.DMA for _ in range(NBUF)],
            [pltpu.SemaphoreType.DMA for _ in range(NBUF)],
            pltpu.VMEM_SHARED((ACC_ROWS, D), jnp.float32),
        ],
    )
    return f(meta_u, meta_i, utab, itab, zeros)


def _prelu(x, a):
    return jnp.where(x >= 0, x, a * x)


def _tc_body(a_ref, ue_ref, ie_ref, uw_ref, iw_ref,
             mu_ref, mi_ref, su_ref, si_ref):
    a = a_ref[0]
    uw = uw_ref[...]
    iw = iw_ref[...]
    yu = []
    yi = []
    for b in range(NB):
        yu.append(jnp.dot(ue_ref[b], uw, preferred_element_type=jnp.float32))
        yi.append(jnp.dot(ie_ref[b], iw, preferred_element_type=jnp.float32))
        su_ref[b] = _prelu(yu[b], a)
        si_ref[b] = _prelu(yi[b], a)
    third = jnp.float32(1.0 / 3.0)
    mu_ref[...] = _prelu((yu[0] + yu[1] + yu[2]) * third, a)
    mi_ref[...] = _prelu((yi[0] + yi[1] + yi[2]) * third, a)


@jax.jit
def _project(ue, ie, u_w, i_w, prelu_a):
    R = 1000
    grid = (U // R,)
    out_shapes = (
        jax.ShapeDtypeStruct((U, D), jnp.float32),
        jax.ShapeDtypeStruct((I, D), jnp.float32),
        jax.ShapeDtypeStruct((NB, U, D), jnp.float32),
        jax.ShapeDtypeStruct((NB, I, D), jnp.float32),
    )
    return pl.pallas_call(
        _tc_body,
        grid=grid,
        in_specs=[
            pl.BlockSpec(memory_space=pltpu.SMEM),
            pl.BlockSpec((NB, R, D), lambda i: (0, i, 0)),
            pl.BlockSpec((NB, R, D), lambda i: (0, i, 0)),
            pl.BlockSpec((D, D), lambda i: (0, 0)),
            pl.BlockSpec((D, D), lambda i: (0, 0)),
        ],
        out_specs=(
            pl.BlockSpec((R, D), lambda i: (i, 0)),
            pl.BlockSpec((R, D), lambda i: (i, 0)),
            pl.BlockSpec((NB, R, D), lambda i: (0, i, 0)),
            pl.BlockSpec((NB, R, D), lambda i: (0, i, 0)),
        ),
        out_shape=out_shapes,
    )(prelu_a.reshape(1), ue, ie, u_w, i_w)


def _pack_table(tab):
    n = tab.shape[0]
    t = tab.reshape(n, D // 32, 2, 16).transpose(0, 1, 3, 2)
    t = t.astype(jnp.bfloat16).reshape(n, D // 2, 2)
    return jax.lax.bitcast_convert_type(t, jnp.int32)


def kernel(init_user_embedding, init_item_embedding, u_w, i_w, prelu_a,
           rows0, cols0, vals0, rows1, cols1, vals1, rows2, cols2, vals2):
    zeros = jnp.zeros((ACC_ROWS, D), jnp.float32)
    rowsc = jnp.concatenate([rows0, rows1, rows2]).reshape(-1, CH)
    colsc = jnp.concatenate([cols0, cols1, cols2]).reshape(-1, CH)
    valsc = lax.bitcast_convert_type(
        jnp.concatenate([vals0, vals1, vals2]), jnp.int32).reshape(-1, CH)
    meta_u = jnp.stack([colsc, rowsc, valsc, valsc], axis=1)
    meta_i = jnp.stack([rowsc, colsc, valsc, valsc], axis=1)
    ue, ie = _spmm_all(meta_u, meta_i,
                       _pack_table(init_user_embedding),
                       _pack_table(init_item_embedding), zeros)
    multi_user, multi_item, single_user, single_item = _project(
        ue, ie, u_w, i_w, prelu_a)
    return (multi_user, multi_item, single_user, single_item)


# speedup vs baseline: 1.4179x; 1.4179x over previous
import jax
import jax.numpy as jnp
from jax import lax
from jax.experimental import pallas as pl
from jax.experimental.pallas import tpu as pltpu
from jax.experimental.pallas import tpu_sc as plsc

U = 5000
I = 5000
D = 128
NNZ = 320000
NB = 3

NC = 2
NS = 16
CH = 80
NCH = NNZ // (NS * CH)
EPT = NNZ // NS
RPT = 320
ACC_ROWS = NS * RPT
NBUF = 5


def _sc_spmm(rowsf_hbm, colsf_hbm, valsf_hbm,
             utab_hbm, itab_hbm, zeros_hbm,
             ue_out, ie_out,
             isrc_bufs, idst_bufs, val_bufs, inbufs, outbufs,
             isems, gsems, ssems, acc):
    c = lax.axis_index("c")
    s = lax.axis_index("s")

    def do_side(src_tab, src_idx_flat, dst_idx_flat, out_hbm):
        def behavior_body(b, carry0):
            pltpu.sync_copy(zeros_hbm.at[pl.ds(s * RPT, RPT)],
                            acc.at[pl.ds(s * RPT, RPT)])
            plsc.subcore_barrier()
            ebase = b * NNZ + s * EPT

            def ifetch_start(i, k):
                off = ebase + i * CH
                pltpu.async_copy(src_idx_flat.at[pl.ds(off, CH)],
                                 isrc_bufs[k], isems[k])
                pltpu.async_copy(dst_idx_flat.at[pl.ds(off, CH)],
                                 idst_bufs[k].at[0], isems[k])
                pltpu.async_copy(valsf_hbm.at[pl.ds(off, CH)],
                                 val_bufs[k].at[pl.ds(0, CH)], isems[k])

            def ifetch_wait(i, k):
                off = ebase + i * CH
                pltpu.make_async_copy(src_idx_flat.at[pl.ds(off, CH)],
                                      isrc_bufs[k], isems[k]).wait()
                pltpu.make_async_copy(dst_idx_flat.at[pl.ds(off, CH)],
                                      idst_bufs[k].at[0], isems[k]).wait()
                pltpu.make_async_copy(valsf_hbm.at[pl.ds(off, CH)],
                                      val_bufs[k].at[pl.ds(0, CH)],
                                      isems[k]).wait()

            def gather_start(k):
                pltpu.async_copy(src_tab.at[isrc_bufs[k]], inbufs[k], gsems[k])

            def gather_wait(k):
                pltpu.make_async_copy(src_tab.at[isrc_bufs[k]],
                                      inbufs[k], gsems[k]).wait()

            def scatter_start(k):
                pltpu.async_copy(outbufs[k], acc.at[idst_bufs[k].at[0]],
                                 ssems[k], add=True)

            def scatter_wait(k):
                pltpu.make_async_copy(outbufs[k], acc.at[idst_bufs[k].at[0]],
                                      ssems[k]).wait()

            def scale(k):
                inb = inbufs[k]
                outb = outbufs[k]
                vbuf = val_bufs[k]
                mask = jnp.int32(-65536)

                @plsc.parallel_loop(0, CH, unroll=5)
                def _(e):
                    val = vbuf[pl.ds(e, 16)][0]
                    for g in range(D // 32):
                        w = inb[e, pl.ds(16 * g, 16)]
                        lo = lax.bitcast_convert_type(
                            lax.shift_left(w, jnp.int32(16)), jnp.float32)
                        hi = lax.bitcast_convert_type(w & mask, jnp.float32)
                        outb[e, pl.ds(32 * g, 16)] = lo * val
                        outb[e, pl.ds(32 * g + 16, 16)] = hi * val

            for k in range(3):
                ifetch_start(k, k)
            for k in range(2):
                ifetch_wait(k, k)
                gather_start(k)

            def chunk_group(g, carry):
                for j in range(NBUF):
                    ch = g * NBUF + j
                    pre = ch + 3
                    kpre = (j + 3) % NBUF
                    mid = ch + 2
                    kmid = (j + 2) % NBUF

                    @pl.when(pre < NCH)
                    def _():
                        @pl.when(pre >= NBUF)
                        def _():
                            scatter_wait(kpre)

                        ifetch_start(pre, kpre)

                    @pl.when(mid < NCH)
                    def _():
                        ifetch_wait(mid, kmid)
                        gather_start(kmid)

                    gather_wait(j)
                    scale(j)
                    scatter_start(j)
                return carry

            lax.fori_loop(0, NCH // NBUF, chunk_group, 0, unroll=False)
            for j in range(NBUF):
                scatter_wait(j)
            plsc.subcore_barrier()

            @pl.when(s < NS - 1)
            def _():
                pltpu.sync_copy(acc.at[pl.ds(s * RPT, RPT)],
                                out_hbm.at[b, pl.ds(s * RPT, RPT)])

            @pl.when(s == NS - 1)
            def _():
                last = U - (NS - 1) * RPT
                pltpu.sync_copy(acc.at[pl.ds((NS - 1) * RPT, last)],
                                out_hbm.at[b, pl.ds((NS - 1) * RPT, last)])

            plsc.subcore_barrier()
            return carry0

        lax.fori_loop(0, NB, behavior_body, 0, unroll=False)

    @pl.when(c == 0)
    def _():
        do_side(itab_hbm, colsf_hbm, rowsf_hbm, ue_out)

    @pl.when(c == 1)
    def _():
        do_side(utab_hbm, rowsf_hbm, colsf_hbm, ie_out)


@jax.jit
def _spmm_all(rowsf, colsf, valsf, utab, itab, zeros):
    mesh = plsc.VectorSubcoreMesh(core_axis_name="c", subcore_axis_name="s",
                                  num_cores=NC, num_subcores=NS)
    f = pl.kernel(
        _sc_spmm,
        out_type=(jax.ShapeDtypeStruct((NB, U, D), jnp.float32),
                  jax.ShapeDtypeStruct((NB, I, D), jnp.float32)),
        mesh=mesh,
        compiler_params=pltpu.CompilerParams(use_tc_tiling_on_sc=False),
        scratch_types=[
            [pltpu.VMEM((CH,), jnp.int32) for _ in range(NBUF)],
            [pltpu.VMEM((1, CH), jnp.int32) for _ in range(NBUF)],
            [pltpu.VMEM((CH + 16,), jnp.float32) for _ in range(NBUF)],
            [pltpu.VMEM((CH, D // 2), jnp.int32) for _ in range(NBUF)],
            [pltpu.VMEM((CH, D), jnp.float32) for _ in range(NBUF)],
            [pltpu.SemaphoreType.DMA for _ in range(NBUF)],
            [pltpu.SemaphoreType.DMA for _ in range(NBUF)],
            [pltpu.SemaphoreType.DMA for _ in range(NBUF)],
            pltpu.VMEM_SHARED((ACC_ROWS, D), jnp.float32),
        ],
    )
    return f(rowsf, colsf, valsf, utab, itab, zeros)


def _prelu(x, a):
    return jnp.where(x >= 0, x, a * x)


def _tc_body(a_ref, ue_ref, ie_ref, uw_ref, iw_ref,
             mu_ref, mi_ref, su_ref, si_ref):
    a = a_ref[0]
    uw = uw_ref[...]
    iw = iw_ref[...]
    yu = []
    yi = []
    for b in range(NB):
        yu.append(jnp.dot(ue_ref[b], uw, preferred_element_type=jnp.float32))
        yi.append(jnp.dot(ie_ref[b], iw, preferred_element_type=jnp.float32))
        su_ref[b] = _prelu(yu[b], a)
        si_ref[b] = _prelu(yi[b], a)
    third = jnp.float32(1.0 / 3.0)
    mu_ref[...] = _prelu((yu[0] + yu[1] + yu[2]) * third, a)
    mi_ref[...] = _prelu((yi[0] + yi[1] + yi[2]) * third, a)


@jax.jit
def _project(ue, ie, u_w, i_w, prelu_a):
    R = 1000
    grid = (U // R,)
    out_shapes = (
        jax.ShapeDtypeStruct((U, D), jnp.float32),
        jax.ShapeDtypeStruct((I, D), jnp.float32),
        jax.ShapeDtypeStruct((NB, U, D), jnp.float32),
        jax.ShapeDtypeStruct((NB, I, D), jnp.float32),
    )
    return pl.pallas_call(
        _tc_body,
        grid=grid,
        in_specs=[
            pl.BlockSpec(memory_space=pltpu.SMEM),
            pl.BlockSpec((NB, R, D), lambda i: (0, i, 0)),
            pl.BlockSpec((NB, R, D), lambda i: (0, i, 0)),
            pl.BlockSpec((D, D), lambda i: (0, 0)),
            pl.BlockSpec((D, D), lambda i: (0, 0)),
        ],
        out_specs=(
            pl.BlockSpec((R, D), lambda i: (i, 0)),
            pl.BlockSpec((R, D), lambda i: (i, 0)),
            pl.BlockSpec((NB, R, D), lambda i: (0, i, 0)),
            pl.BlockSpec((NB, R, D), lambda i: (0, i, 0)),
        ),
        out_shape=out_shapes,
    )(prelu_a.reshape(1), ue, ie, u_w, i_w)


def _pack_table(tab):
    n = tab.shape[0]
    t = tab.reshape(n, D // 32, 2, 16).transpose(0, 1, 3, 2)
    t = t.astype(jnp.bfloat16).reshape(n, D // 2, 2)
    return jax.lax.bitcast_convert_type(t, jnp.int32)


def kernel(init_user_embedding, init_item_embedding, u_w, i_w, prelu_a,
           rows0, cols0, vals0, rows1, cols1, vals1, rows2, cols2, vals2):
    zeros = jnp.zeros((ACC_ROWS, D), jnp.float32)
    rowsf = jnp.concatenate([rows0, rows1, rows2])
    colsf = jnp.concatenate([cols0, cols1, cols2])
    valsf = jnp.concatenate([vals0, vals1, vals2])
    ue, ie = _spmm_all(rowsf, colsf, valsf,
                       _pack_table(init_user_embedding),
                       _pack_table(init_item_embedding), zeros)
    multi_user, multi_item, single_user, single_item = _project(
        ue, ie, u_w, i_w, prelu_a)
    return (multi_user, multi_item, single_user, single_item)
